# Initial kernel scaffold; baseline (speedup 1.0000x reference)
import jax
import jax.numpy as jnp
from jax.experimental import pallas as pl


def kernel(x, edge_index, edge_attr, elem_emb, hyb_emb, W_ep, b_ep, msg_W1, msg_b1, msg_W2, msg_b2, gru_Wih, gru_Whh, gru_bih, gru_bhh, Wp1, bp1, Wp2, bp2, Wp3, bp3):
    raise NotImplementedError("write your pallas kernel here")


# speedup vs baseline: 1.3917x; 1.3917x over previous
import functools

import jax
import jax.numpy as jnp
from jax import lax
from jax.experimental import pallas as pl
from jax.experimental.pallas import tpu as pltpu
from jax.experimental.pallas import tpu_sc as plsc

N = 10000
E = 160000
NP = 10240
EP = 163840
D = 256
EH = 64
H = 256
NL = 3
NC, NS = 2, 16
NW = NC * NS
CH = 128
EPW = EP // NW
EPT = EP // NS
RPT = NP // NS

_F32 = jnp.float32

def _sc_mesh():
    return plsc.VectorSubcoreMesh(
        core_axis_name="c", subcore_axis_name="s",
        num_cores=NC, num_subcores=NS)


def _sc_gather(table, idx):
    def body(table_hbm, idx_hbm, out_hbm, idx_v, rows_v):
        wid = lax.axis_index("s") * NC + lax.axis_index("c")
        base = wid * EPW

        def chunk(j, carry):
            off = base + j * CH
            pltpu.sync_copy(idx_hbm.at[pl.ds(off, CH)], idx_v)
            pltpu.sync_copy(table_hbm.at[idx_v], rows_v)
            pltpu.sync_copy(rows_v, out_hbm.at[pl.ds(off, CH)])
            return carry

        lax.fori_loop(0, EPW // CH, chunk, 0)

    f = pl.kernel(
        body,
        out_type=jax.ShapeDtypeStruct((EP, D), _F32),
        mesh=_sc_mesh(),
        scratch_types=[
            pltpu.VMEM((CH,), jnp.int32),
            pltpu.VMEM((CH, D), _F32),
        ],
    )
    return f(table, idx)


def _sc_scatter_add(m2, dst, zeros_rows):
    def body(m_hbm, dst_hbm, z_hbm, out_hbm, idx_v, rows_v, acc_sh):
        c = lax.axis_index("c")
        s = lax.axis_index("s")
        pltpu.sync_copy(z_hbm, acc_sh.at[pl.ds(s * RPT, RPT)])
        plsc.subcore_barrier()
        ebase = s * EPT

        def chunk(j, carry):
            off = ebase + j * CH
            pltpu.sync_copy(dst_hbm.at[pl.ds(off, CH)], idx_v)
            pltpu.sync_copy(m_hbm.at[pl.ds(c * EP + off, CH)], rows_v)
            pltpu.sync_copy(rows_v, acc_sh.at[idx_v], add=True)
            return carry

        lax.fori_loop(0, EPT // CH, chunk, 0)
        plsc.subcore_barrier()
        pltpu.sync_copy(acc_sh.at[pl.ds(s * RPT, RPT)],
                        out_hbm.at[pl.ds(c * NP + s * RPT, RPT)])

    f = pl.kernel(
        body,
        out_type=jax.ShapeDtypeStruct((2 * NP, 128), _F32),
        mesh=_sc_mesh(),
        scratch_types=[
            pltpu.VMEM((CH,), jnp.int32),
            pltpu.VMEM((CH, 128), _F32),
            pltpu.VMEM_SHARED((NP, 128), _F32),
        ],
    )
    return f(m2, dst, zeros_rows)


def _tc_embed(x0, x1, elem8, hyb8, w1a0):
    BN = 2048

    def body(x0_ref, x1_ref, e8_ref, h8_ref, w_ref, h_ref, hw_ref):
        io = lax.broadcasted_iota(jnp.int32, (BN, 8), 1)
        oh0 = (x0_ref[...] == io).astype(_F32)
        oh1 = (x1_ref[...] == io).astype(_F32)
        h = jnp.concatenate(
            [jnp.dot(oh0, e8_ref[...], preferred_element_type=_F32),
             jnp.dot(oh1, h8_ref[...], preferred_element_type=_F32)], axis=1)
        h_ref[...] = h
        hw_ref[...] = jnp.dot(h, w_ref[...], preferred_element_type=_F32)

    return pl.pallas_call(
        body,
        grid=(NP // BN,),
        in_specs=[
            pl.BlockSpec((BN, 1), lambda i: (i, 0)),
            pl.BlockSpec((BN, 1), lambda i: (i, 0)),
            pl.BlockSpec((8, 128), lambda i: (0, 0)),
            pl.BlockSpec((8, 128), lambda i: (0, 0)),
            pl.BlockSpec((D, D), lambda i: (0, 0)),
        ],
        out_specs=[
            pl.BlockSpec((BN, D), lambda i: (i, 0)),
            pl.BlockSpec((BN, D), lambda i: (i, 0)),
        ],
        out_shape=[jax.ShapeDtypeStruct((NP, D), _F32),
                   jax.ShapeDtypeStruct((NP, D), _F32)],
    )(x0, x1, elem8, hyb8, w1a0)


def _tc_edgeproj(ea, wep, bep):
    BE = 4096

    def body(a_ref, w_ref, b_ref, o_ref):
        o_ref[...] = jnp.maximum(
            jnp.dot(a_ref[...], w_ref[...], preferred_element_type=_F32)
            + b_ref[...], 0.0)

    return pl.pallas_call(
        body,
        grid=(EP // BE,),
        in_specs=[
            pl.BlockSpec((BE, 8), lambda i: (i, 0)),
            pl.BlockSpec((8, EH), lambda i: (0, 0)),
            pl.BlockSpec((1, EH), lambda i: (0, 0)),
        ],
        out_specs=pl.BlockSpec((BE, EH), lambda i: (i, 0)),
        out_shape=jax.ShapeDtypeStruct((EP, EH), _F32),
    )(ea, wep, bep)


def _tc_msg(g, e, w1b, b1, w2, b2):
    BE = 1024

    def body(g_ref, e_ref, w1b_ref, b1_ref, w2_ref, b2_ref, o_ref):
        t = (g_ref[...]
             + jnp.dot(e_ref[...], w1b_ref[...], preferred_element_type=_F32)
             + b1_ref[...])
        t = jnp.maximum(t, 0.0)
        m = jnp.dot(t, w2_ref[...], preferred_element_type=_F32) + b2_ref[...]
        o_ref[0] = m[:, :128]
        o_ref[1] = m[:, 128:]

    return pl.pallas_call(
        body,
        grid=(EP // BE,),
        in_specs=[
            pl.BlockSpec((BE, D), lambda i: (i, 0)),
            pl.BlockSpec((BE, EH), lambda i: (i, 0)),
            pl.BlockSpec((EH, H), lambda i: (0, 0)),
            pl.BlockSpec((1, H), lambda i: (0, 0)),
            pl.BlockSpec((H, H), lambda i: (0, 0)),
            pl.BlockSpec((1, H), lambda i: (0, 0)),
        ],
        out_specs=pl.BlockSpec((2, BE, 128), lambda i: (0, i, 0)),
        out_shape=jax.ShapeDtypeStruct((2, EP, 128), _F32),
    )(g, e, w1b, b1, w2, b2)


def _tc_gru(aggr2, h, wih_t, whh_t, bih, bhh, wproj_a, wproj_b=None):
    BN = 1024
    two = wproj_b is not None

    def body(a_ref, h_ref, wi_ref, wh_ref, bi_ref, bh_ref, wa_ref, *rest):
        if two:
            wb_ref, ho_ref, pa_ref, pb_ref = rest
        else:
            ho_ref, pa_ref = rest
        a = jnp.concatenate([a_ref[0], a_ref[1]], axis=1)
        hh = h_ref[...]
        gi = jnp.dot(a, wi_ref[...], preferred_element_type=_F32) + bi_ref[...]
        gh = jnp.dot(hh, wh_ref[...], preferred_element_type=_F32) + bh_ref[...]
        r = jax.nn.sigmoid(gi[:, :D] + gh[:, :D])
        z = jax.nn.sigmoid(gi[:, D:2 * D] + gh[:, D:2 * D])
        n = jnp.tanh(gi[:, 2 * D:] + r * gh[:, 2 * D:])
        ho = hh + (1.0 - z) * n + z * hh
        ho_ref[...] = ho
        pa_ref[...] = jnp.dot(ho, wa_ref[...], preferred_element_type=_F32)
        if two:
            pb_ref[...] = jnp.dot(ho, wb_ref[...], preferred_element_type=_F32)

    in_specs = [
        pl.BlockSpec((2, BN, 128), lambda i: (0, i, 0)),
        pl.BlockSpec((BN, D), lambda i: (i, 0)),
        pl.BlockSpec((D, 3 * D), lambda i: (0, 0)),
        pl.BlockSpec((D, 3 * D), lambda i: (0, 0)),
        pl.BlockSpec((1, 3 * D), lambda i: (0, 0)),
        pl.BlockSpec((1, 3 * D), lambda i: (0, 0)),
        pl.BlockSpec((D, D), lambda i: (0, 0)),
    ]
    args = [aggr2, h, wih_t, whh_t, bih, bhh, wproj_a]
    nout = 2
    if two:
        in_specs.append(pl.BlockSpec((D, D), lambda i: (0, 0)))
        args.append(wproj_b)
        nout = 3
    return pl.pallas_call(
        body,
        grid=(NP // BN,),
        in_specs=in_specs,
        out_specs=[pl.BlockSpec((BN, D), lambda i: (i, 0))] * nout,
        out_shape=[jax.ShapeDtypeStruct((NP, D), _F32)] * nout,
    )(*args)


def _tc_head(g1, g2, e, wp1c, bp1, wp2, bp2, wp3r, bp3):
    BE = 2048

    def body(g1_ref, g2_ref, e_ref, wc_ref, b1_ref, w2_ref, b2_ref, w3_ref,
             b3_ref, o_ref):
        t = (g1_ref[...] + g2_ref[...]
             + jnp.dot(e_ref[...], wc_ref[...], preferred_element_type=_F32)
             + b1_ref[...])
        t = jnp.maximum(t, 0.0)
        t = jnp.maximum(
            jnp.dot(t, w2_ref[...], preferred_element_type=_F32) + b2_ref[...],
            0.0)
        o_ref[...] = (jnp.sum(t * w3_ref[...], axis=1, keepdims=True)
                      + b3_ref[...])

    return pl.pallas_call(
        body,
        grid=(EP // BE,),
        in_specs=[
            pl.BlockSpec((BE, D), lambda i: (i, 0)),
            pl.BlockSpec((BE, D), lambda i: (i, 0)),
            pl.BlockSpec((BE, EH), lambda i: (i, 0)),
            pl.BlockSpec((EH, H), lambda i: (0, 0)),
            pl.BlockSpec((1, H), lambda i: (0, 0)),
            pl.BlockSpec((H, H // 2), lambda i: (0, 0)),
            pl.BlockSpec((1, H // 2), lambda i: (0, 0)),
            pl.BlockSpec((1, H // 2), lambda i: (0, 0)),
            pl.BlockSpec((1, 1), lambda i: (0, 0)),
        ],
        out_specs=pl.BlockSpec((BE, 1), lambda i: (i, 0)),
        out_shape=jax.ShapeDtypeStruct((EP, 1), _F32),
    )(g1, g2, e, wp1c, bp1, wp2, bp2, wp3r, bp3)


def kernel(x, edge_index, edge_attr, elem_emb, hyb_emb, W_ep, b_ep,
           msg_W1, msg_b1, msg_W2, msg_b2,
           gru_Wih, gru_Whh, gru_bih, gru_bhh,
           Wp1, bp1, Wp2, bp2, Wp3, bp3):
    pad_e = EP - E
    src_p = jnp.concatenate(
        [edge_index[0].astype(jnp.int32), jnp.zeros((pad_e,), jnp.int32)])
    dst_p = jnp.concatenate(
        [edge_index[1].astype(jnp.int32), jnp.full((pad_e,), N, jnp.int32)])
    ea_p = jnp.pad(edge_attr, ((0, pad_e), (0, 2)))
    x0 = jnp.pad(x[:, 0:1].astype(jnp.int32), ((0, NP - N), (0, 0)),
                 constant_values=-1)
    x1 = jnp.pad(x[:, 1:2].astype(jnp.int32), ((0, NP - N), (0, 0)),
                 constant_values=-1)
    wep_p = jnp.pad(W_ep, ((0, 2), (0, 0)))
    w1a = msg_W1[:, :D, :]
    w1b = msg_W1[:, D:, :]
    wih_t = jnp.swapaxes(gru_Wih, 1, 2)
    whh_t = jnp.swapaxes(gru_Whh, 1, 2)
    wp1a, wp1b, wp1c = Wp1[:D], Wp1[D:2 * D], Wp1[2 * D:]
    zeros_rows = jnp.zeros((RPT, 128), _F32)

    h, hw = _tc_embed(x0, x1, elem_emb[:8], hyb_emb[:8], w1a[0])
    e = _tc_edgeproj(ea_p, wep_p, b_ep.reshape(1, EH))
    hp1 = hp2 = None
    for l in range(NL):
        g = _sc_gather(hw, src_p)
        m = _tc_msg(g, e, w1b[l], msg_b1[l].reshape(1, H), msg_W2[l],
                    msg_b2[l].reshape(1, H))
        aggr = _sc_scatter_add(m.reshape(2 * EP, 128), dst_p, zeros_rows)
        aggr2 = aggr.reshape(2, NP, 128)
        bi = gru_bih[l].reshape(1, 3 * D)
        bh = gru_bhh[l].reshape(1, 3 * D)
        if l < NL - 1:
            h, hw = _tc_gru(aggr2, h, wih_t[l], whh_t[l], bi, bh, w1a[l + 1])
        else:
            h, hp1, hp2 = _tc_gru(aggr2, h, wih_t[l], whh_t[l], bi, bh,
                                  wp1a, wp1b)
    g1 = _sc_gather(hp1, src_p)
    g2 = _sc_gather(hp2, dst_p)
    out = _tc_head(g1, g2, e, wp1c, bp1.reshape(1, H), Wp2,
                   bp2.reshape(1, H // 2), Wp3.reshape(1, H // 2),
                   bp3.reshape(1, 1))
    return out[:E, 0]


# speedup vs baseline: 1.7085x; 1.2277x over previous
import functools

import jax
import jax.numpy as jnp
from jax import lax
from jax.experimental import pallas as pl
from jax.experimental.pallas import tpu as pltpu
from jax.experimental.pallas import tpu_sc as plsc

N = 10000
E = 160000
NP = 10240
EP = 163840
D = 256
EH = 64
H = 256
NL = 3
NC, NS = 2, 16
NW = NC * NS
CH = 128
EPW = EP // NW
EPT = EP // NS
RPT = NP // NS

_F32 = jnp.float32

def _sc_mesh():
    return plsc.VectorSubcoreMesh(
        core_axis_name="c", subcore_axis_name="s",
        num_cores=NC, num_subcores=NS)


_NCHG = EPW // CH


def _gather_pipeline(table_hbm, idx3_hbm, out_hbm, wid, idx_all, rows_a,
                     rows_b, gsem_a, gsem_b, ssem_a, ssem_b):
    base = wid * EPW
    pltpu.sync_copy(idx3_hbm.at[wid], idx_all)
    pltpu.async_copy(table_hbm.at[idx_all.at[0]], rows_a, gsem_a)

    @pl.loop(0, _NCHG // 2)
    def _sup(t):
        j = t * 2
        @pl.when(j >= 1)
        def _():
            pltpu.make_async_copy(
                rows_b, out_hbm.at[pl.ds(base + (j - 1) * CH, CH)],
                ssem_b).wait()
        pltpu.async_copy(table_hbm.at[idx_all.at[j + 1]], rows_b, gsem_b)
        pltpu.make_async_copy(table_hbm.at[idx_all.at[j]], rows_a,
                              gsem_a).wait()
        pltpu.async_copy(rows_a, out_hbm.at[pl.ds(base + j * CH, CH)], ssem_a)
        @pl.when(j + 2 < _NCHG)
        def _():
            pltpu.make_async_copy(
                rows_a, out_hbm.at[pl.ds(base + j * CH, CH)], ssem_a).wait()
            pltpu.async_copy(table_hbm.at[idx_all.at[j + 2]], rows_a, gsem_a)
        pltpu.make_async_copy(table_hbm.at[idx_all.at[j + 1]], rows_b,
                              gsem_b).wait()
        pltpu.async_copy(rows_b, out_hbm.at[pl.ds(base + (j + 1) * CH, CH)],
                         ssem_b)

    pltpu.make_async_copy(
        rows_a, out_hbm.at[pl.ds(base + (_NCHG - 2) * CH, CH)], ssem_a).wait()
    pltpu.make_async_copy(
        rows_b, out_hbm.at[pl.ds(base + (_NCHG - 1) * CH, CH)], ssem_b).wait()


def _sc_gather(table, idx3):
    def body(table_hbm, idx3_hbm, out_hbm, idx_all, rows_a, rows_b,
             gsem_a, gsem_b, ssem_a, ssem_b):
        wid = lax.axis_index("s") * NC + lax.axis_index("c")
        _gather_pipeline(table_hbm, idx3_hbm, out_hbm, wid, idx_all,
                         rows_a, rows_b, gsem_a, gsem_b, ssem_a, ssem_b)

    f = pl.kernel(
        body,
        out_type=jax.ShapeDtypeStruct((EP, D), _F32),
        mesh=_sc_mesh(),
        scratch_types=[
            pltpu.VMEM((_NCHG, CH), jnp.int32),
            pltpu.VMEM((CH, D), _F32),
            pltpu.VMEM((CH, D), _F32),
            pltpu.SemaphoreType.DMA,
            pltpu.SemaphoreType.DMA,
            pltpu.SemaphoreType.DMA,
            pltpu.SemaphoreType.DMA,
        ],
    )
    return f(table, idx3)


def _sc_gather2(table1, table2, idx3_1, idx3_2):
    def body(t1_hbm, t2_hbm, i1_hbm, i2_hbm, o1_hbm, o2_hbm, idx_all,
             rows_a, rows_b, gsem_a, gsem_b, ssem_a, ssem_b):
        wid = lax.axis_index("s") * NC + lax.axis_index("c")
        _gather_pipeline(t1_hbm, i1_hbm, o1_hbm, wid, idx_all,
                         rows_a, rows_b, gsem_a, gsem_b, ssem_a, ssem_b)
        _gather_pipeline(t2_hbm, i2_hbm, o2_hbm, wid, idx_all,
                         rows_a, rows_b, gsem_a, gsem_b, ssem_a, ssem_b)

    f = pl.kernel(
        body,
        out_type=[jax.ShapeDtypeStruct((EP, D), _F32),
                  jax.ShapeDtypeStruct((EP, D), _F32)],
        mesh=_sc_mesh(),
        scratch_types=[
            pltpu.VMEM((_NCHG, CH), jnp.int32),
            pltpu.VMEM((CH, D), _F32),
            pltpu.VMEM((CH, D), _F32),
            pltpu.SemaphoreType.DMA,
            pltpu.SemaphoreType.DMA,
            pltpu.SemaphoreType.DMA,
            pltpu.SemaphoreType.DMA,
        ],
    )
    return f(table1, table2, idx3_1, idx3_2)


def _sc_scatter_add(m2, dst, zeros_rows):
    NCHS = EPT // CH

    def body(m_hbm, dst3_hbm, z_hbm, out_hbm, idx_all, rows_a, rows_b,
             lsem_a, lsem_b, ssem_a, ssem_b, acc_sh):
        c = lax.axis_index("c")
        s = lax.axis_index("s")
        pltpu.sync_copy(dst3_hbm.at[s], idx_all)
        pltpu.sync_copy(z_hbm, acc_sh.at[pl.ds(s * RPT, RPT)])
        plsc.subcore_barrier()
        ebase = c * EP + s * EPT

        def m_src(j):
            return m_hbm.at[pl.ds(ebase + j * CH, CH)]

        pltpu.async_copy(m_src(0), rows_a, lsem_a)

        @pl.loop(0, NCHS // 2)
        def _sup(t):
            j = t * 2
            @pl.when(j >= 1)
            def _():
                pltpu.make_async_copy(
                    rows_b, acc_sh.at[idx_all.at[j - 1]], ssem_b).wait()
            pltpu.async_copy(m_src(j + 1), rows_b, lsem_b)
            pltpu.make_async_copy(m_src(j), rows_a, lsem_a).wait()
            pltpu.async_copy(rows_a, acc_sh.at[idx_all.at[j]], ssem_a,
                             add=True)
            @pl.when(j + 2 < NCHS)
            def _():
                pltpu.make_async_copy(
                    rows_a, acc_sh.at[idx_all.at[j]], ssem_a).wait()
                pltpu.async_copy(m_src(j + 2), rows_a, lsem_a)
            pltpu.make_async_copy(m_src(j + 1), rows_b, lsem_b).wait()
            pltpu.async_copy(rows_b, acc_sh.at[idx_all.at[j + 1]], ssem_b,
                             add=True)

        pltpu.make_async_copy(
            rows_a, acc_sh.at[idx_all.at[NCHS - 2]], ssem_a).wait()
        pltpu.make_async_copy(
            rows_b, acc_sh.at[idx_all.at[NCHS - 1]], ssem_b).wait()
        plsc.subcore_barrier()
        pltpu.sync_copy(acc_sh.at[pl.ds(s * RPT, RPT)],
                        out_hbm.at[pl.ds(c * NP + s * RPT, RPT)])

    f = pl.kernel(
        body,
        out_type=jax.ShapeDtypeStruct((2 * NP, 128), _F32),
        mesh=_sc_mesh(),
        scratch_types=[
            pltpu.VMEM((EPT // CH, CH), jnp.int32),
            pltpu.VMEM((CH, 128), _F32),
            pltpu.VMEM((CH, 128), _F32),
            pltpu.SemaphoreType.DMA,
            pltpu.SemaphoreType.DMA,
            pltpu.SemaphoreType.DMA,
            pltpu.SemaphoreType.DMA,
            pltpu.VMEM_SHARED((NP, 128), _F32),
        ],
    )
    return f(m2, dst, zeros_rows)


def _tc_embed(x0, x1, elem8, hyb8, w1a0):
    BN = 2048

    def body(x0_ref, x1_ref, e8_ref, h8_ref, w_ref, h_ref, hw_ref):
        io = lax.broadcasted_iota(jnp.int32, (BN, 8), 1)
        oh0 = (x0_ref[...] == io).astype(_F32)
        oh1 = (x1_ref[...] == io).astype(_F32)
        h = jnp.concatenate(
            [jnp.dot(oh0, e8_ref[...], preferred_element_type=_F32),
             jnp.dot(oh1, h8_ref[...], preferred_element_type=_F32)], axis=1)
        h_ref[...] = h
        hw_ref[...] = jnp.dot(h, w_ref[...], preferred_element_type=_F32)

    return pl.pallas_call(
        body,
        grid=(NP // BN,),
        in_specs=[
            pl.BlockSpec((BN, 1), lambda i: (i, 0)),
            pl.BlockSpec((BN, 1), lambda i: (i, 0)),
            pl.BlockSpec((8, 128), lambda i: (0, 0)),
            pl.BlockSpec((8, 128), lambda i: (0, 0)),
            pl.BlockSpec((D, D), lambda i: (0, 0)),
        ],
        out_specs=[
            pl.BlockSpec((BN, D), lambda i: (i, 0)),
            pl.BlockSpec((BN, D), lambda i: (i, 0)),
        ],
        out_shape=[jax.ShapeDtypeStruct((NP, D), _F32),
                   jax.ShapeDtypeStruct((NP, D), _F32)],
    )(x0, x1, elem8, hyb8, w1a0)


def _tc_edgeproj(ea, wep, bep):
    BE = 4096

    def body(a_ref, w_ref, b_ref, o_ref):
        o_ref[...] = jnp.maximum(
            jnp.dot(a_ref[...], w_ref[...], preferred_element_type=_F32)
            + b_ref[...], 0.0)

    return pl.pallas_call(
        body,
        grid=(EP // BE,),
        in_specs=[
            pl.BlockSpec((BE, 8), lambda i: (i, 0)),
            pl.BlockSpec((8, EH), lambda i: (0, 0)),
            pl.BlockSpec((1, EH), lambda i: (0, 0)),
        ],
        out_specs=pl.BlockSpec((BE, EH), lambda i: (i, 0)),
        out_shape=jax.ShapeDtypeStruct((EP, EH), _F32),
    )(ea, wep, bep)


def _tc_msg(g, e, w1b, b1, w2, b2):
    BE = 1024

    def body(g_ref, e_ref, w1b_ref, b1_ref, w2_ref, b2_ref, o_ref):
        t = (g_ref[...]
             + jnp.dot(e_ref[...], w1b_ref[...], preferred_element_type=_F32)
             + b1_ref[...])
        t = jnp.maximum(t, 0.0)
        m = jnp.dot(t, w2_ref[...], preferred_element_type=_F32) + b2_ref[...]
        o_ref[0] = m[:, :128]
        o_ref[1] = m[:, 128:]

    return pl.pallas_call(
        body,
        grid=(EP // BE,),
        in_specs=[
            pl.BlockSpec((BE, D), lambda i: (i, 0)),
            pl.BlockSpec((BE, EH), lambda i: (i, 0)),
            pl.BlockSpec((EH, H), lambda i: (0, 0)),
            pl.BlockSpec((1, H), lambda i: (0, 0)),
            pl.BlockSpec((H, H), lambda i: (0, 0)),
            pl.BlockSpec((1, H), lambda i: (0, 0)),
        ],
        out_specs=pl.BlockSpec((2, BE, 128), lambda i: (0, i, 0)),
        out_shape=jax.ShapeDtypeStruct((2, EP, 128), _F32),
    )(g, e, w1b, b1, w2, b2)


def _tc_gru(aggr2, h, wih_t, whh_t, bih, bhh, wproj_a, wproj_b=None):
    BN = 1024
    two = wproj_b is not None

    def body(a_ref, h_ref, wi_ref, wh_ref, bi_ref, bh_ref, wa_ref, *rest):
        if two:
            wb_ref, ho_ref, pa_ref, pb_ref = rest
        else:
            ho_ref, pa_ref = rest
        a = jnp.concatenate([a_ref[0], a_ref[1]], axis=1)
        hh = h_ref[...]
        gi = jnp.dot(a, wi_ref[...], preferred_element_type=_F32) + bi_ref[...]
        gh = jnp.dot(hh, wh_ref[...], preferred_element_type=_F32) + bh_ref[...]
        r = jax.nn.sigmoid(gi[:, :D] + gh[:, :D])
        z = jax.nn.sigmoid(gi[:, D:2 * D] + gh[:, D:2 * D])
        n = jnp.tanh(gi[:, 2 * D:] + r * gh[:, 2 * D:])
        ho = hh + (1.0 - z) * n + z * hh
        ho_ref[...] = ho
        pa_ref[...] = jnp.dot(ho, wa_ref[...], preferred_element_type=_F32)
        if two:
            pb_ref[...] = jnp.dot(ho, wb_ref[...], preferred_element_type=_F32)

    in_specs = [
        pl.BlockSpec((2, BN, 128), lambda i: (0, i, 0)),
        pl.BlockSpec((BN, D), lambda i: (i, 0)),
        pl.BlockSpec((D, 3 * D), lambda i: (0, 0)),
        pl.BlockSpec((D, 3 * D), lambda i: (0, 0)),
        pl.BlockSpec((1, 3 * D), lambda i: (0, 0)),
        pl.BlockSpec((1, 3 * D), lambda i: (0, 0)),
        pl.BlockSpec((D, D), lambda i: (0, 0)),
    ]
    args = [aggr2, h, wih_t, whh_t, bih, bhh, wproj_a]
    nout = 2
    if two:
        in_specs.append(pl.BlockSpec((D, D), lambda i: (0, 0)))
        args.append(wproj_b)
        nout = 3
    return pl.pallas_call(
        body,
        grid=(NP // BN,),
        in_specs=in_specs,
        out_specs=[pl.BlockSpec((BN, D), lambda i: (i, 0))] * nout,
        out_shape=[jax.ShapeDtypeStruct((NP, D), _F32)] * nout,
    )(*args)


def _tc_head(g1, g2, e, wp1c, bp1, wp2, bp2, wp3r, bp3):
    BE = 2048

    def body(g1_ref, g2_ref, e_ref, wc_ref, b1_ref, w2_ref, b2_ref, w3_ref,
             b3_ref, o_ref):
        t = (g1_ref[...] + g2_ref[...]
             + jnp.dot(e_ref[...], wc_ref[...], preferred_element_type=_F32)
             + b1_ref[...])
        t = jnp.maximum(t, 0.0)
        t = jnp.maximum(
            jnp.dot(t, w2_ref[...], preferred_element_type=_F32) + b2_ref[...],
            0.0)
        o_ref[...] = (jnp.sum(t * w3_ref[...], axis=1, keepdims=True)
                      + b3_ref[...])

    return pl.pallas_call(
        body,
        grid=(EP // BE,),
        in_specs=[
            pl.BlockSpec((BE, D), lambda i: (i, 0)),
            pl.BlockSpec((BE, D), lambda i: (i, 0)),
            pl.BlockSpec((BE, EH), lambda i: (i, 0)),
            pl.BlockSpec((EH, H), lambda i: (0, 0)),
            pl.BlockSpec((1, H), lambda i: (0, 0)),
            pl.BlockSpec((H, H // 2), lambda i: (0, 0)),
            pl.BlockSpec((1, H // 2), lambda i: (0, 0)),
            pl.BlockSpec((1, H // 2), lambda i: (0, 0)),
            pl.BlockSpec((1, 1), lambda i: (0, 0)),
        ],
        out_specs=pl.BlockSpec((BE, 1), lambda i: (i, 0)),
        out_shape=jax.ShapeDtypeStruct((EP, 1), _F32),
    )(g1, g2, e, wp1c, bp1, wp2, bp2, wp3r, bp3)


def kernel(x, edge_index, edge_attr, elem_emb, hyb_emb, W_ep, b_ep,
           msg_W1, msg_b1, msg_W2, msg_b2,
           gru_Wih, gru_Whh, gru_bih, gru_bhh,
           Wp1, bp1, Wp2, bp2, Wp3, bp3):
    pad_e = EP - E
    src_p = jnp.concatenate(
        [edge_index[0].astype(jnp.int32), jnp.zeros((pad_e,), jnp.int32)])
    dst_p = jnp.concatenate(
        [edge_index[1].astype(jnp.int32), jnp.full((pad_e,), N, jnp.int32)])
    ea_p = jnp.pad(edge_attr, ((0, pad_e), (0, 2)))
    x0 = jnp.pad(x[:, 0:1].astype(jnp.int32), ((0, NP - N), (0, 0)),
                 constant_values=-1)
    x1 = jnp.pad(x[:, 1:2].astype(jnp.int32), ((0, NP - N), (0, 0)),
                 constant_values=-1)
    wep_p = jnp.pad(W_ep, ((0, 2), (0, 0)))
    w1a = msg_W1[:, :D, :]
    w1b = msg_W1[:, D:, :]
    wih_t = jnp.swapaxes(gru_Wih, 1, 2)
    whh_t = jnp.swapaxes(gru_Whh, 1, 2)
    wp1a, wp1b, wp1c = Wp1[:D], Wp1[D:2 * D], Wp1[2 * D:]
    zeros_rows = jnp.zeros((RPT, 128), _F32)
    src_g3 = src_p.reshape(NW, _NCHG, CH)
    dst_g3 = dst_p.reshape(NW, _NCHG, CH)
    dst_s3 = dst_p.reshape(NS, EPT // CH, CH)

    h, hw = _tc_embed(x0, x1, elem_emb[:8], hyb_emb[:8], w1a[0])
    e = _tc_edgeproj(ea_p, wep_p, b_ep.reshape(1, EH))
    hp1 = hp2 = None
    for l in range(NL):
        g = _sc_gather(hw, src_g3)
        m = _tc_msg(g, e, w1b[l], msg_b1[l].reshape(1, H), msg_W2[l],
                    msg_b2[l].reshape(1, H))
        aggr = _sc_scatter_add(m.reshape(2 * EP, 128), dst_s3, zeros_rows)
        aggr2 = aggr.reshape(2, NP, 128)
        bi = gru_bih[l].reshape(1, 3 * D)
        bh = gru_bhh[l].reshape(1, 3 * D)
        if l < NL - 1:
            h, hw = _tc_gru(aggr2, h, wih_t[l], whh_t[l], bi, bh, w1a[l + 1])
        else:
            h, hp1, hp2 = _tc_gru(aggr2, h, wih_t[l], whh_t[l], bi, bh,
                                  wp1a, wp1b)
    g1, g2 = _sc_gather2(hp1, hp2, src_g3, dst_g3)
    out = _tc_head(g1, g2, e, wp1c, bp1.reshape(1, H), Wp2,
                   bp2.reshape(1, H // 2), Wp3.reshape(1, H // 2),
                   bp3.reshape(1, 1))
    return out[:E, 0]
